# Initial kernel scaffold; baseline (speedup 1.0000x reference)
import jax
import jax.numpy as jnp
from jax.experimental import pallas as pl


def kernel(h, edge_index, W, a_w):
    raise NotImplementedError("write your pallas kernel here")


# speedup vs baseline: 18.5658x; 18.5658x over previous
import functools

import jax
import jax.numpy as jnp
from jax import lax
from jax.experimental import pallas as pl
from jax.experimental.pallas import tpu as pltpu
from jax.experimental.pallas import tpu_sc as plsc

N = 10000
E = 320000
D = 128
NPAD = 10240
NTILES = 32
EPT = E // NTILES
CHUNK = 128
NFULL = EPT // CHUNK
TAIL = EPT - NFULL * CHUNK
RPT = NPAD // 16


def _tc_pre(h_ref, wt_ref, ap_ref, z_ref, s_ref):
    z = jnp.dot(h_ref[...], wt_ref[...], preferred_element_type=jnp.float32)
    z_ref[...] = z
    s_ref[...] = jnp.dot(z, ap_ref[...], preferred_element_type=jnp.float32)


def _tc_post(acc_ref, den_ref, out_ref):
    p = acc_ref[0] + acc_ref[1]
    den = jnp.sum(den_ref[...], axis=0)[:, None]
    out_ref[...] = jnp.where(den > 0.0, p / den, 0.0)


def _sc_kernel(src_hbm, dst_hbm, s1_hbm, s2_hbm, z_hbm, zero_hbm, zero1_hbm,
               acc_out, den_out,
               s1_v, s2_v, den_v, src_idx, dst_idx, src_idx_t, dst_idx_t,
               w_buf, rows_v, sem, acc_sh):
    cid = lax.axis_index("c")
    sid = lax.axis_index("s")
    g = sid * 2 + cid

    pltpu.sync_copy(s1_hbm, s1_v)
    pltpu.sync_copy(s2_hbm, s2_v)
    pltpu.sync_copy(zero1_hbm, den_v)
    pltpu.sync_copy(zero_hbm.at[pl.ds(sid * RPT, RPT)],
                    acc_sh.at[pl.ds(sid * RPT, RPT)])
    plsc.subcore_barrier()

    ebase = g * EPT

    def process(eb, n, s_idx, d_idx):
        pltpu.sync_copy(src_hbm.at[pl.ds(eb, n)], s_idx)
        pltpu.sync_copy(dst_hbm.at[pl.ds(eb, n)], d_idx)
        gcp = pltpu.async_copy(z_hbm.at[s_idx], rows_v.at[pl.ds(0, n)], sem)

        def wgrp(k, carry):
            sv = s_idx[pl.ds(k * 16, 16)]
            dv = d_idx[pl.ds(k * 16, 16)]
            e = plsc.load_gather(s1_v, [sv]) + plsc.load_gather(s2_v, [dv])
            e = jnp.maximum(e, e * 0.01)
            w = jnp.exp(e)
            w_buf[pl.ds(k * 16, 16)] = w
            plsc.addupdate_scatter(den_v, [dv], w)
            return carry

        lax.fori_loop(0, n // 16, wgrp, 0)
        gcp.wait()

        def scale(i, carry):
            wv = plsc.load_gather(w_buf, [jnp.zeros((16,), jnp.int32) + i])
            for j in range(8):
                rows_v[i, pl.ds(j * 16, 16)] = rows_v[i, pl.ds(j * 16, 16)] * wv
            return carry

        lax.fori_loop(0, n, scale, 0)

    def chunk(i, carry):
        process(ebase + i * CHUNK, CHUNK, src_idx, dst_idx)
        pltpu.sync_copy(rows_v, acc_sh.at[dst_idx], add=True)
        return carry

    lax.fori_loop(0, NFULL, chunk, 0)

    process(ebase + NFULL * CHUNK, TAIL, src_idx_t, dst_idx_t)
    pltpu.sync_copy(rows_v.at[pl.ds(0, TAIL)], acc_sh.at[dst_idx_t], add=True)

    pltpu.sync_copy(den_v, den_out.at[g])
    plsc.subcore_barrier()
    pltpu.sync_copy(acc_sh.at[pl.ds(sid * RPT, RPT)],
                    acc_out.at[cid, pl.ds(sid * RPT, RPT)])


def kernel(h, edge_index, W, a_w):
    src = edge_index[0]
    dst = edge_index[1]
    h_pad = jnp.concatenate(
        [h, jnp.zeros((NPAD - N, D), jnp.float32)], axis=0)
    wt = W.T
    ap = jnp.zeros((D, D), jnp.float32)
    ap = ap.at[:, 0].set(a_w[0, :D]).at[:, 1].set(a_w[0, D:])

    z, s = pl.pallas_call(
        _tc_pre,
        out_shape=(jax.ShapeDtypeStruct((NPAD, D), jnp.float32),
                   jax.ShapeDtypeStruct((NPAD, D), jnp.float32)),
    )(h_pad, wt, ap)
    s1 = s[:, 0]
    s2 = s[:, 1]

    mesh = plsc.VectorSubcoreMesh(core_axis_name="c", subcore_axis_name="s")
    sc = pl.kernel(
        _sc_kernel,
        out_type=(jax.ShapeDtypeStruct((2, NPAD, D), jnp.float32),
                  jax.ShapeDtypeStruct((NTILES, NPAD), jnp.float32)),
        mesh=mesh,
        compiler_params=pltpu.CompilerParams(needs_layout_passes=False),
        scratch_types=[
            pltpu.VMEM((NPAD,), jnp.float32),
            pltpu.VMEM((NPAD,), jnp.float32),
            pltpu.VMEM((NPAD,), jnp.float32),
            pltpu.VMEM((CHUNK,), jnp.int32),
            pltpu.VMEM((CHUNK,), jnp.int32),
            pltpu.VMEM((TAIL,), jnp.int32),
            pltpu.VMEM((TAIL,), jnp.int32),
            pltpu.VMEM((CHUNK,), jnp.float32),
            pltpu.VMEM((CHUNK, D), jnp.float32),
            pltpu.SemaphoreType.DMA,
            pltpu.VMEM_SHARED((NPAD, D), jnp.float32),
        ],
    )
    zero = jnp.zeros((NPAD, D), jnp.float32)
    zero1 = jnp.zeros((NPAD,), jnp.float32)
    acc, dens = sc(src, dst, s1, s2, z, zero, zero1)

    h_out = pl.pallas_call(
        _tc_post,
        out_shape=jax.ShapeDtypeStruct((NPAD, D), jnp.float32),
    )(acc, dens)
    return h_out[:N]


# speedup vs baseline: 20.4267x; 1.1002x over previous
import functools

import jax
import jax.numpy as jnp
from jax import lax
from jax.experimental import pallas as pl
from jax.experimental.pallas import tpu as pltpu
from jax.experimental.pallas import tpu_sc as plsc

N = 10000
E = 320000
D = 128
NTILES = 32
EPT = E // NTILES
CHUNK = 64
NFULL = EPT // CHUNK
TAIL = EPT - NFULL * CHUNK
NPAD = 10240
RPT = NPAD // 16


def _tc_pre(h_ref, wt_ref, ap_ref, z_ref, s_ref):
    z = jnp.dot(h_ref[...], wt_ref[...], preferred_element_type=jnp.float32)
    z_ref[...] = z
    s_ref[...] = jnp.dot(z, ap_ref[...], preferred_element_type=jnp.float32)


def _tc_post(acc_ref, den_ref, out_ref):
    p = acc_ref[0, :N] + acc_ref[1, :N]
    den = jnp.sum(den_ref[...], axis=0)[:, None]
    out_ref[...] = jnp.where(den > 0.0, p / den, 0.0)


def _sc_kernel(src_hbm, dst_hbm, s1_hbm, s2_hbm, z_hbm, zero_hbm, zero1_hbm,
               acc_out, den_out,
               s1_v, s2_v, den_v,
               src_idx0, dst_idx0, src_idx1, dst_idx1,
               src_idx_t, dst_idx_t,
               w_buf, rows0, rows1, sem0, sem1, acc_sh):
    cid = lax.axis_index("c")
    sid = lax.axis_index("s")
    g = sid * 2 + cid

    sidx = (src_idx0, src_idx1)
    didx = (dst_idx0, dst_idx1)
    rws = (rows0, rows1)
    sms = (sem0, sem1)

    pltpu.sync_copy(s1_hbm, s1_v)
    pltpu.sync_copy(s2_hbm, s2_v)
    pltpu.sync_copy(zero1_hbm, den_v)
    pltpu.sync_copy(zero_hbm.at[pl.ds(sid * RPT, RPT)],
                    acc_sh.at[pl.ds(sid * RPT, RPT)])
    plsc.subcore_barrier()

    ebase = g * EPT

    def fetch(b, c):
        eb = ebase + c * CHUNK
        pltpu.sync_copy(src_hbm.at[pl.ds(eb, CHUNK)], sidx[b])
        pltpu.sync_copy(dst_hbm.at[pl.ds(eb, CHUNK)], didx[b])
        pltpu.async_copy(z_hbm.at[sidx[b]], rws[b], sms[b])

    def logits(s_idx, d_idx, n):
        for k in range(n // 16):
            sv = s_idx[pl.ds(k * 16, 16)]
            dv = d_idx[pl.ds(k * 16, 16)]
            e = plsc.load_gather(s1_v, [sv]) + plsc.load_gather(s2_v, [dv])
            e = jnp.maximum(e, e * 0.01)
            w = jnp.exp(e)
            w_buf[pl.ds(k * 16, 16)] = w
            plsc.addupdate_scatter(den_v, [dv], w)

    def scale_rows(rows, n):
        def scale(i, carry):
            wv = plsc.load_gather(w_buf, [jnp.zeros((16,), jnp.int32) + i])
            for j in range(8):
                rows[i, pl.ds(j * 16, 16)] = rows[i, pl.ds(j * 16, 16)] * wv
            return carry
        lax.fori_loop(0, n, scale, 0, unroll=4)

    fetch(0, 0)

    @pl.loop(0, NFULL, step=2)
    def _(i):
        for b in range(2):
            c = i + b

            @pl.when(c < NFULL - 1)
            def _():
                fetch(1 - b, c + 1)

            logits(sidx[b], didx[b], CHUNK)
            pltpu.make_async_copy(z_hbm.at[sidx[b]], rws[b], sms[b]).wait()
            scale_rows(rws[b], CHUNK)
            pltpu.sync_copy(rws[b], acc_sh.at[didx[b]], add=True)

    eb = ebase + NFULL * CHUNK
    pltpu.sync_copy(src_hbm.at[pl.ds(eb, TAIL)], src_idx_t)
    pltpu.sync_copy(dst_hbm.at[pl.ds(eb, TAIL)], dst_idx_t)
    pltpu.async_copy(z_hbm.at[src_idx_t], rows0.at[pl.ds(0, TAIL)], sem0)
    logits(src_idx_t, dst_idx_t, TAIL)
    pltpu.make_async_copy(
        z_hbm.at[src_idx_t], rows0.at[pl.ds(0, TAIL)], sem0).wait()
    scale_rows(rows0, TAIL)
    pltpu.sync_copy(rows0.at[pl.ds(0, TAIL)], acc_sh.at[dst_idx_t], add=True)

    pltpu.sync_copy(den_v, den_out.at[g])
    plsc.subcore_barrier()
    pltpu.sync_copy(acc_sh.at[pl.ds(sid * RPT, RPT)],
                    acc_out.at[cid, pl.ds(sid * RPT, RPT)])


def kernel(h, edge_index, W, a_w):
    src = edge_index[0]
    dst = edge_index[1]
    wt = W.T
    ap = jnp.zeros((D, D), jnp.float32)
    ap = ap.at[:, 0].set(a_w[0, :D]).at[:, 1].set(a_w[0, D:])

    z, s = pl.pallas_call(
        _tc_pre,
        out_shape=(jax.ShapeDtypeStruct((N, D), jnp.float32),
                   jax.ShapeDtypeStruct((N, D), jnp.float32)),
    )(h, wt, ap)
    s1 = s[:, 0]
    s2 = s[:, 1]

    mesh = plsc.VectorSubcoreMesh(core_axis_name="c", subcore_axis_name="s")
    sc = pl.kernel(
        _sc_kernel,
        out_type=(jax.ShapeDtypeStruct((2, NPAD, D), jnp.float32),
                  jax.ShapeDtypeStruct((NTILES, N), jnp.float32)),
        mesh=mesh,
        compiler_params=pltpu.CompilerParams(needs_layout_passes=False),
        scratch_types=[
            pltpu.VMEM((N,), jnp.float32),
            pltpu.VMEM((N,), jnp.float32),
            pltpu.VMEM((N,), jnp.float32),
            pltpu.VMEM((CHUNK,), jnp.int32),
            pltpu.VMEM((CHUNK,), jnp.int32),
            pltpu.VMEM((CHUNK,), jnp.int32),
            pltpu.VMEM((CHUNK,), jnp.int32),
            pltpu.VMEM((TAIL,), jnp.int32),
            pltpu.VMEM((TAIL,), jnp.int32),
            pltpu.VMEM((CHUNK,), jnp.float32),
            pltpu.VMEM((CHUNK, D), jnp.float32),
            pltpu.VMEM((CHUNK, D), jnp.float32),
            pltpu.SemaphoreType.DMA,
            pltpu.SemaphoreType.DMA,
            pltpu.VMEM_SHARED((NPAD, D), jnp.float32),
        ],
    )
    zero = jnp.zeros((NPAD, D), jnp.float32)
    zero1 = jnp.zeros((N,), jnp.float32)
    acc, dens = sc(src, dst, s1, s2, z, zero, zero1)

    h_out = pl.pallas_call(
        _tc_post,
        out_shape=jax.ShapeDtypeStruct((N, D), jnp.float32),
    )(acc, dens)
    return h_out
